# baseline (speedup 1.0000x reference)
import functools

import jax
import jax.numpy as jnp
from jax import lax
from jax.experimental import pallas as pl
from jax.experimental.pallas import tpu as pltpu
from jax.experimental.pallas import tpu_sc as plsc

_SCALAR = 20.0
_LOG2E = 1.4426950408889634
_LN2 = 0.6931471805599453


def _sc_gather(lut, targets):
    k_, d_ = lut.shape
    b_ = targets.shape[0]
    info = plsc.get_sparse_core_info()
    nw = info.num_cores * info.num_subcores
    b_per_w = b_ // nw
    mesh = plsc.VectorSubcoreMesh(core_axis_name="c", subcore_axis_name="s")

    @functools.partial(
        pl.kernel,
        mesh=mesh,
        compiler_params=pltpu.CompilerParams(use_tc_tiling_on_sc=False),
        out_type=jax.ShapeDtypeStruct((b_, d_), jnp.float32),
        scratch_types=[
            pltpu.VMEM((b_per_w,), jnp.int32),
            pltpu.VMEM((b_per_w, d_), jnp.float32),
            pltpu.SemaphoreType.DMA,
        ],
    )
    def gather_kernel(table_hbm, idx_hbm, out_hbm, idx_v, rows_v, sem):
        wid = lax.axis_index("s") * info.num_cores + lax.axis_index("c")
        base = wid * b_per_w
        pltpu.sync_copy(idx_hbm.at[pl.ds(base, b_per_w)], idx_v)
        pltpu.async_copy(table_hbm.at[idx_v], rows_v, sem).wait()
        pltpu.sync_copy(rows_v, out_hbm.at[pl.ds(base, b_per_w)])

    return gather_kernel(lut, targets)


def _lse_loss_body(x_ref, lutt_ref, rows_ref, out_ref, m_ref, s_ref, *, k_total, tk):
    kstep = pl.program_id(0)
    nk = pl.num_programs(0)

    @pl.when(kstep == 0)
    def _init():
        m_ref[...] = jnp.full_like(m_ref, -jnp.inf)
        s_ref[...] = jnp.zeros_like(s_ref)

    x = x_ref[...]
    xb = (x * (_SCALAR * _LOG2E)).astype(jnp.bfloat16)
    t = lax.dot_general(
        xb, lutt_ref[...],
        dimension_numbers=(((1,), (0,)), ((), ())),
        preferred_element_type=jnp.float32,
    ).astype(jnp.bfloat16)

    def _mask_tail(tt):
        col = kstep * tk + lax.broadcasted_iota(jnp.int32, tt.shape, 1)
        return jnp.where(col < k_total, tt, jnp.bfloat16(-jnp.inf))

    t = lax.cond(kstep == nk - 1, _mask_tail, lambda tt: tt, t)

    m_old = m_ref[...]
    m_new = jnp.maximum(m_old, jnp.max(t, axis=1, keepdims=True).astype(jnp.float32))
    p = jnp.exp2(t - m_new.astype(jnp.bfloat16))
    p_sum = jnp.sum(p, axis=1, keepdims=True).astype(jnp.float32)
    s_ref[...] = s_ref[...] * jnp.exp2(m_old - m_new) + p_sum
    m_ref[...] = m_new

    @pl.when(kstep == nk - 1)
    def _finish():
        picked = _SCALAR * jnp.sum(x * rows_ref[...], axis=1, keepdims=True)
        lse = (m_ref[...] + jnp.log2(s_ref[...])) * _LN2
        out_ref[...] = jnp.sum(lse - picked, axis=0, keepdims=True) / x.shape[0]


def _tc_lse_loss(inputs, lut_t, rows, tk=4096):
    b_, d_ = inputs.shape
    k_ = lut_t.shape[1]
    nk = pl.cdiv(k_, tk)
    out = pl.pallas_call(
        functools.partial(_lse_loss_body, k_total=k_, tk=tk),
        grid=(nk,),
        in_specs=[
            pl.BlockSpec((b_, d_), lambda k: (0, 0)),
            pl.BlockSpec((d_, tk), lambda k: (0, k)),
            pl.BlockSpec((b_, d_), lambda k: (0, 0)),
        ],
        out_specs=pl.BlockSpec((1, 1), lambda k: (0, 0)),
        out_shape=jax.ShapeDtypeStruct((1, 1), jnp.float32),
        scratch_shapes=[
            pltpu.VMEM((b_, 1), jnp.float32),
            pltpu.VMEM((b_, 1), jnp.float32),
        ],
    )(inputs, lut_t, rows)
    return out[0, 0]


def kernel(inputs, targets, lut):
    rows = _sc_gather(lut, targets)
    lut_t = lut.T.astype(jnp.bfloat16)
    return _tc_lse_loss(inputs, lut_t, rows)


# speedup vs baseline: 1.3108x; 1.3108x over previous
import functools

import jax
import jax.numpy as jnp
from jax import lax
from jax.experimental import pallas as pl
from jax.experimental.pallas import tpu as pltpu
from jax.experimental.pallas import tpu_sc as plsc

_SCALAR = 20.0
_LOG2E = 1.4426950408889634
_LN2 = 0.6931471805599453


def _sc_gather(lut, targets):
    k_, d_ = lut.shape
    b_ = targets.shape[0]
    info = plsc.get_sparse_core_info()
    nw = info.num_cores * info.num_subcores
    b_per_w = b_ // nw
    mesh = plsc.VectorSubcoreMesh(core_axis_name="c", subcore_axis_name="s")

    @functools.partial(
        pl.kernel,
        mesh=mesh,
        compiler_params=pltpu.CompilerParams(use_tc_tiling_on_sc=False),
        out_type=jax.ShapeDtypeStruct((b_, d_), jnp.float32),
        scratch_types=[
            pltpu.VMEM((b_per_w,), jnp.int32),
            pltpu.VMEM((b_per_w, d_), jnp.float32),
            pltpu.SemaphoreType.DMA,
        ],
    )
    def gather_kernel(table_hbm, idx_hbm, out_hbm, idx_v, rows_v, sem):
        wid = lax.axis_index("s") * info.num_cores + lax.axis_index("c")
        base = wid * b_per_w
        pltpu.sync_copy(idx_hbm.at[pl.ds(base, b_per_w)], idx_v)
        pltpu.async_copy(table_hbm.at[idx_v], rows_v, sem).wait()
        pltpu.sync_copy(rows_v, out_hbm.at[pl.ds(base, b_per_w)])

    return gather_kernel(lut, targets)


def _lse_loss_body(x_ref, lut_hbm, rows_ref, out_ref, m_ref, s_ref, buf, sem,
                   *, tk):
    kstep = pl.program_id(0)
    nk = pl.num_programs(0)
    slot = lax.rem(kstep, 2)
    nslot = lax.rem(kstep + 1, 2)

    @pl.when(kstep == 0)
    def _init():
        m_ref[...] = jnp.full_like(m_ref, -jnp.inf)
        s_ref[...] = jnp.zeros_like(s_ref)
        pltpu.make_async_copy(
            lut_hbm.at[pl.ds(0, tk), :], buf.at[0], sem.at[0]).start()

    @pl.when(kstep + 1 < nk)
    def _prefetch():
        pltpu.make_async_copy(
            lut_hbm.at[pl.ds((kstep + 1) * tk, tk), :], buf.at[nslot],
            sem.at[nslot]).start()

    pltpu.make_async_copy(
        lut_hbm.at[pl.ds(kstep * tk, tk), :], buf.at[slot],
        sem.at[slot]).wait()

    x = x_ref[...]
    xb = (x * (_SCALAR * _LOG2E)).astype(jnp.bfloat16)
    t = lax.dot_general(
        xb, buf[slot].astype(jnp.bfloat16),
        dimension_numbers=(((1,), (1,)), ((), ())),
        preferred_element_type=jnp.float32,
    ).astype(jnp.bfloat16)

    m_old = m_ref[...]
    m_new = jnp.maximum(m_old, jnp.max(t, axis=1, keepdims=True).astype(jnp.float32))
    p = jnp.exp2(t - m_new.astype(jnp.bfloat16))
    p_sum = jnp.sum(p, axis=1, keepdims=True).astype(jnp.float32)
    s_ref[...] = s_ref[...] * jnp.exp2(m_old - m_new) + p_sum
    m_ref[...] = m_new

    @pl.when(kstep == nk - 1)
    def _finish():
        picked = _SCALAR * jnp.sum(x * rows_ref[...], axis=1, keepdims=True)
        lse = (m_ref[...] + jnp.log2(s_ref[...])) * _LN2
        out_ref[...] = jnp.sum(lse - picked, axis=0, keepdims=True) / x.shape[0]


def _tc_lse_loss(inputs, lut, rows, tk=4000):
    b_, d_ = inputs.shape
    k_ = lut.shape[0]
    nk = k_ // tk
    out = pl.pallas_call(
        functools.partial(_lse_loss_body, tk=tk),
        grid=(nk,),
        in_specs=[
            pl.BlockSpec((b_, d_), lambda k: (0, 0)),
            pl.BlockSpec(memory_space=pl.ANY),
            pl.BlockSpec((b_, d_), lambda k: (0, 0)),
        ],
        out_specs=pl.BlockSpec((1, 1), lambda k: (0, 0)),
        out_shape=jax.ShapeDtypeStruct((1, 1), jnp.float32),
        scratch_shapes=[
            pltpu.VMEM((b_, 1), jnp.float32),
            pltpu.VMEM((b_, 1), jnp.float32),
            pltpu.VMEM((2, tk, d_), jnp.float32),
            pltpu.SemaphoreType.DMA((2,)),
        ],
    )(inputs, lut, rows)
    return out[0, 0]


def kernel(inputs, targets, lut):
    rows = _sc_gather(lut, targets)
    return _tc_lse_loss(inputs, lut, rows)
